# baseline (speedup 1.0000x reference)
import functools

import jax
import jax.numpy as jnp
from jax import lax
from jax.experimental import pallas as pl
from jax.experimental.pallas import tpu as pltpu
from jax.experimental.pallas import tpu_sc as plsc

N = 10000
E = 160000
F_IN = 256
HID = 512
G = 64
NP = 10240

NC, NS, L = 2, 16, 16
NW = NC * NS

_MESH = dict(core_axis_name="c", subcore_axis_name="s", num_cores=NC,
             num_subcores=NS)
_CP = pltpu.CompilerParams(needs_layout_passes=False)

F32 = jnp.float32
I32 = jnp.int32

_CH = 4096
_NCHUNK = 40
_EPAD = _CH * _NCHUNK
_CAP = _CH + 64


def _make_reduce(F, RS, nrange, gather, BK):
  npass = nrange // NW
  ACR = RS + 2
  NV = _CH // L

  scratch = [
      pltpu.VMEM((_CH,), I32),
      pltpu.VMEM((_CAP,), I32),
      pltpu.VMEM((ACR * F,), F32),
  ]
  if gather:
    scratch += [
        pltpu.VMEM((_CH,), I32),
        pltpu.VMEM((_CAP,), I32),
        pltpu.VMEM((BK, F), F32),
        pltpu.SemaphoreType.DMA,
    ]

  mesh = plsc.VectorSubcoreMesh(**_MESH)

  @functools.partial(
      pl.kernel,
      out_type=jax.ShapeDtypeStruct((nrange * RS * F,), F32),
      mesh=mesh,
      compiler_params=_CP,
      scratch_types=scratch,
  )
  def reduce_kernel(src_hbm, dst_hbm, t_hbm, out_hbm, dstv, relc, acc, *rest):
    s = lax.axis_index("s")
    c = lax.axis_index("c")
    w = c * NS + s
    if gather:
      srcv, srcc, rows, sem = rest
    iota = lax.iota(I32, L)
    ones = jnp.ones((L,), F32)

    def one_pass(p, carry0):
      base = (w + NW * p) * RS

      def zero(i, carry):
        acc[pl.ds(i * L, L)] = jnp.zeros((L,), F32)
        return carry

      lax.fori_loop(0, ACR * F // L, zero, jnp.int32(0))

      def chunk_body(ch, carry):
        pltpu.sync_copy(dst_hbm.at[pl.ds(ch * _CH, _CH)], dstv)
        if gather:
          pltpu.sync_copy(src_hbm.at[pl.ds(ch * _CH, _CH)], srcv)

        def compact(i, cnt):
          d = dstv[pl.ds(i * L, L)]
          rel = d - base
          m = (rel >= 0) & (rel < RS)
          nhit = plsc.all_reduce_population_count(m)[0]

          def dostore(cnt):
            mi = jnp.where(m, jnp.int32(1), jnp.int32(0))
            pos = cnt + plsc.cumsum(mi) - mi
            plsc.store_scatter(relc, [pos], rel, mask=m)
            if gather:
              sv = srcv[pl.ds(i * L, L)]
              plsc.store_scatter(srcc, [pos], sv, mask=m)

          pl.when(nhit > 0)(lambda: dostore(cnt))
          return cnt + nhit

        cnt = lax.fori_loop(0, NV, compact, jnp.int32(0))

        if not gather:
          nb = (cnt + (BK - 1)) // BK
          for l in range(BK // L):
            lanes = cnt + l * L + iota
            m2 = lanes < nb * BK
            plsc.store_scatter(relc, [lanes], jnp.full((L,), RS, I32),
                               mask=m2)

          def proc(j, carry2):
            def onegrp(g, carry3):
              rv = relc[pl.ds(j * BK + g * L, L)]
              for l in range(L):
                plsc.addupdate(acc.at[pl.ds(rv[l] * 16, L)], ones)
              return carry3
            return lax.fori_loop(0, BK // L, onegrp, carry2)

          lax.fori_loop(0, nb, proc, jnp.int32(0))
          return carry

        nb = (cnt + (BK - 1)) // BK
        for l in range(BK // L):
          lanes = cnt + l * L + iota
          m2 = lanes < nb * BK
          plsc.store_scatter(relc, [lanes], jnp.full((L,), RS, I32), mask=m2)
          plsc.store_scatter(srcc, [lanes], jnp.zeros((L,), I32), mask=m2)

        def proc(j, carry2):
          pltpu.async_copy(t_hbm.at[srcc.at[pl.ds(j * BK, BK)]], rows,
                           sem).wait()

          def onegrp(g, carry3):
            rv = relc[pl.ds(j * BK + g * L, L)]
            for l in range(L):
              roff = rv[l] * F
              for k in range(0, F // L, 4):
                rr = [rows[g * L + l, pl.ds((k + u) * L, L)]
                      for u in range(4)]
                aa = [acc[pl.ds(roff + (k + u) * L, L)] for u in range(4)]
                for u in range(4):
                  acc[pl.ds(roff + (k + u) * L, L)] = aa[u] + rr[u]
            return carry3

          lax.fori_loop(0, BK // L, onegrp, jnp.int32(0))
          return carry2

        lax.fori_loop(0, nb, proc, jnp.int32(0))
        return carry

      lax.fori_loop(0, _NCHUNK, chunk_body, jnp.int32(0))
      pltpu.sync_copy(acc.at[pl.ds(0, RS * F)],
                      out_hbm.at[pl.ds(base * F, RS * F)])
      return carry0

    lax.fori_loop(0, npass, one_pass, jnp.int32(0))

  return reduce_kernel


_RS_DEG, _NR_DEG = 328, 32
_RS1, _NR1 = 328, 32
_RS2, _NR2 = 168, 64

_deg_red = _make_reduce(16, _RS_DEG, _NR_DEG, gather=False, BK=16)
_agg_fin = _make_reduce(F_IN, _RS1, _NR1, gather=True, BK=64)
_agg_hid = _make_reduce(HID, _RS2, _NR2, gather=True, BK=48)


_BM = 512
_GRID = NP // _BM


def _dinv_of(p_ref):
  deg = p_ref[:, 0:1] + 1.0
  return lax.rsqrt(deg)


def _tc_scale(p_ref, x_ref, o_ref):
  o_ref[...] = x_ref[...] * _dinv_of(p_ref)


def _tc_layer(a_ref, t_ref, p_ref, w_ref, b_ref, o_ref):
  dinv = _dinv_of(p_ref)
  u = (a_ref[...] + t_ref[...]) * dinv
  h = jnp.dot(u, w_ref[...], preferred_element_type=F32,
              precision=lax.Precision.HIGHEST)
  h = jnp.maximum(h + b_ref[...], 0.0)
  o_ref[...] = h * dinv


def _tc_pool(a_ref, t_ref, p_ref, w_ref, b_ref, batch_ref, o_ref,
             pooled, cnt2):
  i = pl.program_id(0)

  @pl.when(i == 0)
  def _():
    pooled[...] = jnp.zeros_like(pooled)
    cnt2[...] = jnp.zeros_like(cnt2)

  dinv = _dinv_of(p_ref)
  u = (a_ref[...] + t_ref[...]) * dinv
  h = jnp.dot(u, w_ref[...], preferred_element_type=F32,
              precision=lax.Precision.HIGHEST)
  h = jnp.maximum(h + b_ref[...], 0.0)

  gids = lax.broadcasted_iota(I32, (G, _BM), 0)
  oh = (batch_ref[0] == gids).astype(F32)
  pooled[...] += jnp.dot(oh, h, preferred_element_type=F32,
                         precision=lax.Precision.HIGHEST)
  cnt2[...] += oh

  @pl.when(i == _GRID - 1)
  def _():
    counts = jnp.sum(cnt2[...], axis=1, keepdims=True)
    o_ref[...] = pooled[...] / jnp.maximum(counts, 1.0)


def _row_spec(wd):
  return pl.BlockSpec((_BM, wd), lambda i: (i, 0))


def _full_spec(r, wd):
  return pl.BlockSpec((r, wd), lambda i: (0, 0))


def kernel(x, edge_index, batch, W1, b1, W2, b2):
  src = edge_index[0]
  dst = edge_index[1]
  srcp = jnp.pad(src, (0, _EPAD - E))
  dstp = jnp.pad(dst, (0, _EPAD - E), constant_values=NP)

  x_pad = jnp.pad(x, ((0, NP - N), (0, 0)))
  batch2 = jnp.pad(batch, (0, NP - N),
                   constant_values=G).reshape(_GRID, 1, _BM)
  b1r = b1.reshape(1, HID)
  b2r = b2.reshape(1, HID)
  dummy16 = jnp.zeros((1, 16), F32)

  p = _deg_red(srcp, dstp, dummy16).reshape(_NR_DEG * _RS_DEG, 16)

  t1 = pl.pallas_call(
      _tc_scale,
      grid=(_GRID,),
      in_specs=[_row_spec(16), _row_spec(F_IN)],
      out_specs=_row_spec(F_IN),
      out_shape=jax.ShapeDtypeStruct((NP, F_IN), F32),
  )(p, x_pad)

  a1 = _agg_fin(srcp, dstp, t1).reshape(_NR1 * _RS1, F_IN)

  t2 = pl.pallas_call(
      _tc_layer,
      grid=(_GRID,),
      in_specs=[_row_spec(F_IN), _row_spec(F_IN), _row_spec(16),
                _full_spec(F_IN, HID), _full_spec(1, HID)],
      out_specs=_row_spec(HID),
      out_shape=jax.ShapeDtypeStruct((NP, HID), F32),
  )(a1, t1, p, W1, b1r)

  a2 = _agg_hid(srcp, dstp, t2).reshape(_NR2 * _RS2, HID)

  out = pl.pallas_call(
      _tc_pool,
      grid=(_GRID,),
      in_specs=[_row_spec(HID), _row_spec(HID), _row_spec(16),
                _full_spec(HID, HID), _full_spec(1, HID),
                pl.BlockSpec((1, 1, _BM), lambda i: (i, 0, 0))],
      out_specs=_full_spec(G, HID),
      out_shape=jax.ShapeDtypeStruct((G, HID), F32),
      scratch_shapes=[pltpu.VMEM((G, HID), F32), pltpu.VMEM((G, _BM), F32)],
  )(a2, t2, p, W2, b2r, batch2)

  return out


# speedup vs baseline: 2.0226x; 2.0226x over previous
import functools

import jax
import jax.numpy as jnp
from jax import lax
from jax.experimental import pallas as pl
from jax.experimental.pallas import tpu as pltpu
from jax.experimental.pallas import tpu_sc as plsc

N = 10000
E = 160000
F_IN = 256
HID = 512
G = 64
NP = 10240

NC, NS, L = 2, 16, 16
NW = NC * NS

_MESH = dict(core_axis_name="c", subcore_axis_name="s", num_cores=NC,
             num_subcores=NS)
_CP = pltpu.CompilerParams(needs_layout_passes=False)

F32 = jnp.float32
I32 = jnp.int32

_CH = 4096
_NCHUNK = 40
_EPAD = _CH * _NCHUNK
_CAP = _CH + 128


def _make_reduce(F, RS, nrange, gather, BK):
  npass = nrange // NW
  ACR = RS + 2
  NV = _CH // L

  scratch = [
      pltpu.VMEM((_CH,), I32),
      pltpu.VMEM((_CAP,), I32),
      pltpu.VMEM((ACR * F,), F32),
  ]
  if gather:
    scratch += [
        pltpu.VMEM((_CH,), I32),
        pltpu.VMEM((_CAP,), I32),
        pltpu.VMEM((BK, F), F32),
        pltpu.SemaphoreType.DMA,
    ]

  mesh = plsc.VectorSubcoreMesh(**_MESH)

  @functools.partial(
      pl.kernel,
      out_type=jax.ShapeDtypeStruct((nrange * RS * F,), F32),
      mesh=mesh,
      compiler_params=_CP,
      scratch_types=scratch,
  )
  def reduce_kernel(src_hbm, dst_hbm, t_hbm, out_hbm, dstv, relc, acc, *rest):
    s = lax.axis_index("s")
    c = lax.axis_index("c")
    w = c * NS + s
    if gather:
      srcv, srcc, rows, sem = rest
    iota = lax.iota(I32, L)
    ones = jnp.ones((L,), F32)

    def one_pass(p, carry0):
      base = (w + NW * p) * RS

      def zero(i, carry):
        acc[pl.ds(i * L, L)] = jnp.zeros((L,), F32)
        return carry

      lax.fori_loop(0, ACR * F // L, zero, jnp.int32(0))

      def chunk_body(ch, cnt_in):
        pltpu.sync_copy(dst_hbm.at[pl.ds(ch * _CH, _CH)], dstv)
        if gather:
          pltpu.sync_copy(src_hbm.at[pl.ds(ch * _CH, _CH)], srcv)

        def compact(i, cnt):
          d = dstv[pl.ds(i * L, L)]
          rel = d - base
          m = (rel >= 0) & (rel < RS)
          nhit = plsc.all_reduce_population_count(m)[0]

          def dostore(cnt):
            mi = jnp.where(m, jnp.int32(1), jnp.int32(0))
            pos = cnt + plsc.cumsum(mi) - mi
            plsc.store_scatter(relc, [pos], rel, mask=m)
            if gather:
              sv = srcv[pl.ds(i * L, L)]
              plsc.store_scatter(srcc, [pos], sv, mask=m)

          pl.when(nhit > 0)(lambda: dostore(cnt))
          return cnt + nhit

        cnt = lax.fori_loop(0, NV, compact, cnt_in)

        if not gather:
          nb = (cnt + (BK - 1)) // BK
          for l in range(BK // L):
            lanes = cnt + l * L + iota
            m2 = lanes < nb * BK
            plsc.store_scatter(relc, [lanes], jnp.full((L,), RS, I32),
                               mask=m2)

          def proc(j, carry2):
            def onegrp(g, carry3):
              rv = relc[pl.ds(j * BK + g * L, L)]
              for l in range(L):
                plsc.addupdate(acc.at[pl.ds(rv[l] * 16, L)], ones)
              return carry3
            return lax.fori_loop(0, BK // L, onegrp, carry2)

          lax.fori_loop(0, nb, proc, jnp.int32(0))
          return jnp.int32(0)

        nfull = cnt // BK
        _proc_blocks(relc, srcc, acc, rows, sem, t_hbm, jnp.int32(0), nfull)
        left = cnt - nfull * BK
        for l in range(BK // L):
          lanes = l * L + iota
          m2 = lanes < left
          vals_r = relc[pl.ds(nfull * BK + l * L, L)]
          vals_s = srcc[pl.ds(nfull * BK + l * L, L)]
          plsc.store_scatter(relc, [lanes], vals_r, mask=m2)
          plsc.store_scatter(srcc, [lanes], vals_s, mask=m2)
        return left

      def _proc_blocks(relc, srcc, acc, rows, sem, t_hbm, lo, hi):
        def proc(j, carry2):
          pltpu.async_copy(t_hbm.at[srcc.at[pl.ds(j * BK, BK)]], rows,
                           sem).wait()

          def onegrp(g, carry3):
            rv = relc[pl.ds(j * BK + g * L, L)]
            for l in range(L):
              roff = rv[l] * F
              for k in range(0, F // L, 4):
                rr = [rows[g * L + l, pl.ds((k + u) * L, L)]
                      for u in range(4)]
                aa = [acc[pl.ds(roff + (k + u) * L, L)] for u in range(4)]
                for u in range(4):
                  acc[pl.ds(roff + (k + u) * L, L)] = aa[u] + rr[u]
            return carry3

          lax.fori_loop(0, BK // L, onegrp, jnp.int32(0))
          return carry2

        lax.fori_loop(lo, hi, proc, jnp.int32(0))

      cnt_last = lax.fori_loop(0, _NCHUNK, chunk_body, jnp.int32(0))
      if gather:
        nb2 = (cnt_last + (BK - 1)) // BK
        for l in range(BK // L):
          lanes = cnt_last + l * L + iota
          m2 = lanes < nb2 * BK
          plsc.store_scatter(relc, [lanes], jnp.full((L,), RS, I32),
                             mask=m2)
          plsc.store_scatter(srcc, [lanes], jnp.zeros((L,), I32), mask=m2)
        _proc_blocks(relc, srcc, acc, rows, sem, t_hbm, jnp.int32(0), nb2)
      pltpu.sync_copy(acc.at[pl.ds(0, RS * F)],
                      out_hbm.at[pl.ds(base * F, RS * F)])
      return carry0

    lax.fori_loop(0, npass, one_pass, jnp.int32(0))

  return reduce_kernel


_RS_DEG, _NR_DEG = 328, 32
_RS1, _NR1 = 328, 32
_RS2, _NR2 = 168, 64

_deg_red = _make_reduce(16, _RS_DEG, _NR_DEG, gather=False, BK=16)
_agg_fin = _make_reduce(F_IN, _RS1, _NR1, gather=True, BK=32)
_agg_hid = _make_reduce(HID, _RS2, _NR2, gather=True, BK=32)


_BM = 512
_GRID = NP // _BM


def _dinv_of(p_ref):
  deg = p_ref[:, 0:1] + 1.0
  return lax.rsqrt(deg)


def _tc_scale(p_ref, x_ref, o_ref):
  o_ref[...] = x_ref[...] * _dinv_of(p_ref)


def _tc_layer(a_ref, t_ref, p_ref, w_ref, b_ref, o_ref):
  dinv = _dinv_of(p_ref)
  u = (a_ref[...] + t_ref[...]) * dinv
  h = jnp.dot(u, w_ref[...], preferred_element_type=F32,
              precision=lax.Precision.HIGHEST)
  h = jnp.maximum(h + b_ref[...], 0.0)
  o_ref[...] = h * dinv


def _tc_pool(a_ref, t_ref, p_ref, w_ref, b_ref, batch_ref, o_ref,
             pooled, cnt2):
  i = pl.program_id(0)

  @pl.when(i == 0)
  def _():
    pooled[...] = jnp.zeros_like(pooled)
    cnt2[...] = jnp.zeros_like(cnt2)

  dinv = _dinv_of(p_ref)
  u = (a_ref[...] + t_ref[...]) * dinv
  h = jnp.dot(u, w_ref[...], preferred_element_type=F32,
              precision=lax.Precision.HIGHEST)
  h = jnp.maximum(h + b_ref[...], 0.0)

  gids = lax.broadcasted_iota(I32, (G, _BM), 0)
  oh = (batch_ref[0] == gids).astype(F32)
  pooled[...] += jnp.dot(oh, h, preferred_element_type=F32,
                         precision=lax.Precision.HIGHEST)
  cnt2[...] += oh

  @pl.when(i == _GRID - 1)
  def _():
    counts = jnp.sum(cnt2[...], axis=1, keepdims=True)
    o_ref[...] = pooled[...] / jnp.maximum(counts, 1.0)


def _row_spec(wd):
  return pl.BlockSpec((_BM, wd), lambda i: (i, 0))


def _full_spec(r, wd):
  return pl.BlockSpec((r, wd), lambda i: (0, 0))


def kernel(x, edge_index, batch, W1, b1, W2, b2):
  src = edge_index[0]
  dst = edge_index[1]
  srcp = jnp.pad(src, (0, _EPAD - E))
  dstp = jnp.pad(dst, (0, _EPAD - E), constant_values=NP)

  x_pad = jnp.pad(x, ((0, NP - N), (0, 0)))
  batch2 = jnp.pad(batch, (0, NP - N),
                   constant_values=G).reshape(_GRID, 1, _BM)
  b1r = b1.reshape(1, HID)
  b2r = b2.reshape(1, HID)
  dummy16 = jnp.zeros((1, 16), F32)

  p = _deg_red(srcp, dstp, dummy16).reshape(_NR_DEG * _RS_DEG, 16)

  t1 = pl.pallas_call(
      _tc_scale,
      grid=(_GRID,),
      in_specs=[_row_spec(16), _row_spec(F_IN)],
      out_specs=_row_spec(F_IN),
      out_shape=jax.ShapeDtypeStruct((NP, F_IN), F32),
  )(p, x_pad)

  a1 = _agg_fin(srcp, dstp, t1).reshape(_NR1 * _RS1, F_IN)

  t2 = pl.pallas_call(
      _tc_layer,
      grid=(_GRID,),
      in_specs=[_row_spec(F_IN), _row_spec(F_IN), _row_spec(16),
                _full_spec(F_IN, HID), _full_spec(1, HID)],
      out_specs=_row_spec(HID),
      out_shape=jax.ShapeDtypeStruct((NP, HID), F32),
  )(a1, t1, p, W1, b1r)

  a2 = _agg_hid(srcp, dstp, t2).reshape(_NR2 * _RS2, HID)

  out = pl.pallas_call(
      _tc_pool,
      grid=(_GRID,),
      in_specs=[_row_spec(HID), _row_spec(HID), _row_spec(16),
                _full_spec(HID, HID), _full_spec(1, HID),
                pl.BlockSpec((1, 1, _BM), lambda i: (i, 0, 0))],
      out_specs=_full_spec(G, HID),
      out_shape=jax.ShapeDtypeStruct((G, HID), F32),
      scratch_shapes=[pltpu.VMEM((G, HID), F32), pltpu.VMEM((G, _BM), F32)],
  )(a2, t2, p, W2, b2r, batch2)

  return out


# speedup vs baseline: 2.0548x; 1.0159x over previous
import functools

import jax
import jax.numpy as jnp
from jax import lax
from jax.experimental import pallas as pl
from jax.experimental.pallas import tpu as pltpu
from jax.experimental.pallas import tpu_sc as plsc

N = 10000
E = 160000
F_IN = 256
HID = 512
G = 64
NP = 10240

NC, NS, L = 2, 16, 16
NW = NC * NS

_MESH = dict(core_axis_name="c", subcore_axis_name="s", num_cores=NC,
             num_subcores=NS)
_CP = pltpu.CompilerParams(needs_layout_passes=False)

F32 = jnp.float32
I32 = jnp.int32

_CH = 4096
_NCHUNK = 40
_EPAD = _CH * _NCHUNK
_CAP = _CH + 128


def _make_reduce(F, RS, nrange, gather, BK):
  npass = nrange // NW
  ACR = RS + 2
  NV = _CH // L

  scratch = [
      pltpu.VMEM((_CH,), I32),
      pltpu.VMEM((_CAP,), I32),
      pltpu.VMEM((ACR * F,), F32),
  ]
  if gather:
    scratch += [
        pltpu.VMEM((_CH,), I32),
        pltpu.VMEM((_CAP,), I32),
        pltpu.VMEM((BK, F), F32),
        pltpu.SemaphoreType.DMA,
    ]

  mesh = plsc.VectorSubcoreMesh(**_MESH)

  @functools.partial(
      pl.kernel,
      out_type=jax.ShapeDtypeStruct((nrange * RS * F,), F32),
      mesh=mesh,
      compiler_params=_CP,
      scratch_types=scratch,
  )
  def reduce_kernel(src_hbm, dst_hbm, t_hbm, out_hbm, dstv, relc, acc, *rest):
    s = lax.axis_index("s")
    c = lax.axis_index("c")
    w = c * NS + s
    if gather:
      srcv, srcc, rows, sem = rest
    iota = lax.iota(I32, L)
    ones = jnp.ones((L,), F32)

    def one_pass(p, carry0):
      base = (w + NW * p) * RS

      def zero(i, carry):
        acc[pl.ds(i * L, L)] = jnp.zeros((L,), F32)
        return carry

      lax.fori_loop(0, ACR * F // L, zero, jnp.int32(0))

      def chunk_body(ch, cnt_in):
        pltpu.sync_copy(dst_hbm.at[pl.ds(ch * _CH, _CH)], dstv)
        if gather:
          pltpu.sync_copy(src_hbm.at[pl.ds(ch * _CH, _CH)], srcv)

        def compact(i, cnt):
          d = dstv[pl.ds(i * L, L)]
          rel = d - base
          m = (rel >= 0) & (rel < RS)
          nhit = plsc.all_reduce_population_count(m)[0]

          def dostore(cnt):
            mi = jnp.where(m, jnp.int32(1), jnp.int32(0))
            pos = cnt + plsc.cumsum(mi) - mi
            plsc.store_scatter(relc, [pos], rel, mask=m)
            if gather:
              sv = srcv[pl.ds(i * L, L)]
              plsc.store_scatter(srcc, [pos], sv, mask=m)

          pl.when(nhit > 0)(lambda: dostore(cnt))
          return cnt + nhit

        cnt = lax.fori_loop(0, NV, compact, cnt_in)

        if not gather:
          nb = (cnt + (BK - 1)) // BK
          for l in range(BK // L):
            lanes = cnt + l * L + iota
            m2 = lanes < nb * BK
            plsc.store_scatter(relc, [lanes], jnp.full((L,), RS, I32),
                               mask=m2)

          def proc(j, carry2):
            def onegrp(g, carry3):
              rv = relc[pl.ds(j * BK + g * L, L)]
              for l in range(L):
                plsc.addupdate(acc.at[pl.ds(rv[l] * 16, L)], ones)
              return carry3
            return lax.fori_loop(0, BK // L, onegrp, carry2)

          lax.fori_loop(0, nb, proc, jnp.int32(0))
          return jnp.int32(0)

        nfull = cnt // BK
        _proc_blocks(relc, srcc, acc, rows, sem, t_hbm, jnp.int32(0), nfull)
        left = cnt - nfull * BK
        for l in range(BK // L):
          lanes = l * L + iota
          m2 = lanes < left
          vals_r = relc[pl.ds(nfull * BK + l * L, L)]
          vals_s = srcc[pl.ds(nfull * BK + l * L, L)]
          plsc.store_scatter(relc, [lanes], vals_r, mask=m2)
          plsc.store_scatter(srcc, [lanes], vals_s, mask=m2)
        return left

      def _proc_blocks(relc, srcc, acc, rows, sem, t_hbm, lo, hi):
        def proc(j, carry2):
          pltpu.async_copy(t_hbm.at[srcc.at[pl.ds(j * BK, BK)]], rows,
                           sem).wait()

          def onegrp(g, carry3):
            rv = relc[pl.ds(j * BK + g * L, L)]
            for l in range(L):
              roff = rv[l] * F
              for k in range(0, F // L, 4):
                rr = [rows[g * L + l, pl.ds((k + u) * L, L)]
                      for u in range(4)]
                aa = [acc[pl.ds(roff + (k + u) * L, L)] for u in range(4)]
                for u in range(4):
                  acc[pl.ds(roff + (k + u) * L, L)] = aa[u] + rr[u]
            return carry3

          lax.fori_loop(0, BK // L, onegrp, jnp.int32(0))
          return carry2

        lax.fori_loop(lo, hi, proc, jnp.int32(0))

      cnt_last = lax.fori_loop(0, _NCHUNK, chunk_body, jnp.int32(0))
      if gather:
        nb2 = (cnt_last + (BK - 1)) // BK
        for l in range(BK // L):
          lanes = cnt_last + l * L + iota
          m2 = lanes < nb2 * BK
          plsc.store_scatter(relc, [lanes], jnp.full((L,), RS, I32),
                             mask=m2)
          plsc.store_scatter(srcc, [lanes], jnp.zeros((L,), I32), mask=m2)
        _proc_blocks(relc, srcc, acc, rows, sem, t_hbm, jnp.int32(0), nb2)
      pltpu.sync_copy(acc.at[pl.ds(0, RS * F)],
                      out_hbm.at[pl.ds(base * F, RS * F)])
      return carry0

    lax.fori_loop(0, npass, one_pass, jnp.int32(0))

  return reduce_kernel


_RS_DEG, _NR_DEG = 328, 32
_RS1, _NR1 = 328, 32
_RS2, _NR2 = 168, 64

_deg_red = _make_reduce(16, _RS_DEG, _NR_DEG, gather=False, BK=16)
_agg_fin = _make_reduce(F_IN, _RS1, _NR1, gather=True, BK=64)
_agg_hid = _make_reduce(HID, _RS2, _NR2, gather=True, BK=48)


_BM = 512
_GRID = NP // _BM


def _dinv_of(p_ref):
  deg = p_ref[:, 0:1] + 1.0
  return lax.rsqrt(deg)


def _tc_scale(p_ref, x_ref, o_ref):
  o_ref[...] = x_ref[...] * _dinv_of(p_ref)


def _tc_layer(a_ref, t_ref, p_ref, w_ref, b_ref, o_ref):
  dinv = _dinv_of(p_ref)
  u = (a_ref[...] + t_ref[...]) * dinv
  h = jnp.dot(u, w_ref[...], preferred_element_type=F32,
              precision=lax.Precision.HIGHEST)
  h = jnp.maximum(h + b_ref[...], 0.0)
  o_ref[...] = h * dinv


def _tc_pool(a_ref, t_ref, p_ref, w_ref, b_ref, batch_ref, o_ref,
             pooled, cnt2):
  i = pl.program_id(0)

  @pl.when(i == 0)
  def _():
    pooled[...] = jnp.zeros_like(pooled)
    cnt2[...] = jnp.zeros_like(cnt2)

  dinv = _dinv_of(p_ref)
  u = (a_ref[...] + t_ref[...]) * dinv
  h = jnp.dot(u, w_ref[...], preferred_element_type=F32,
              precision=lax.Precision.HIGHEST)
  h = jnp.maximum(h + b_ref[...], 0.0)

  gids = lax.broadcasted_iota(I32, (G, _BM), 0)
  oh = (batch_ref[0] == gids).astype(F32)
  pooled[...] += jnp.dot(oh, h, preferred_element_type=F32,
                         precision=lax.Precision.HIGHEST)
  cnt2[...] += oh

  @pl.when(i == _GRID - 1)
  def _():
    counts = jnp.sum(cnt2[...], axis=1, keepdims=True)
    o_ref[...] = pooled[...] / jnp.maximum(counts, 1.0)


def _row_spec(wd):
  return pl.BlockSpec((_BM, wd), lambda i: (i, 0))


def _full_spec(r, wd):
  return pl.BlockSpec((r, wd), lambda i: (0, 0))


def kernel(x, edge_index, batch, W1, b1, W2, b2):
  src = edge_index[0]
  dst = edge_index[1]
  srcp = jnp.pad(src, (0, _EPAD - E))
  dstp = jnp.pad(dst, (0, _EPAD - E), constant_values=NP)

  x_pad = jnp.pad(x, ((0, NP - N), (0, 0)))
  batch2 = jnp.pad(batch, (0, NP - N),
                   constant_values=G).reshape(_GRID, 1, _BM)
  b1r = b1.reshape(1, HID)
  b2r = b2.reshape(1, HID)
  dummy16 = jnp.zeros((1, 16), F32)

  p = _deg_red(srcp, dstp, dummy16).reshape(_NR_DEG * _RS_DEG, 16)

  t1 = pl.pallas_call(
      _tc_scale,
      grid=(_GRID,),
      in_specs=[_row_spec(16), _row_spec(F_IN)],
      out_specs=_row_spec(F_IN),
      out_shape=jax.ShapeDtypeStruct((NP, F_IN), F32),
  )(p, x_pad)

  a1 = _agg_fin(srcp, dstp, t1).reshape(_NR1 * _RS1, F_IN)

  t2 = pl.pallas_call(
      _tc_layer,
      grid=(_GRID,),
      in_specs=[_row_spec(F_IN), _row_spec(F_IN), _row_spec(16),
                _full_spec(F_IN, HID), _full_spec(1, HID)],
      out_specs=_row_spec(HID),
      out_shape=jax.ShapeDtypeStruct((NP, HID), F32),
  )(a1, t1, p, W1, b1r)

  a2 = _agg_hid(srcp, dstp, t2).reshape(_NR2 * _RS2, HID)

  out = pl.pallas_call(
      _tc_pool,
      grid=(_GRID,),
      in_specs=[_row_spec(HID), _row_spec(HID), _row_spec(16),
                _full_spec(HID, HID), _full_spec(1, HID),
                pl.BlockSpec((1, 1, _BM), lambda i: (i, 0, 0))],
      out_specs=_full_spec(G, HID),
      out_shape=jax.ShapeDtypeStruct((G, HID), F32),
      scratch_shapes=[pltpu.VMEM((G, HID), F32), pltpu.VMEM((G, _BM), F32)],
  )(a2, t2, p, W2, b2r, batch2)

  return out


# speedup vs baseline: 2.1597x; 1.0510x over previous
import functools

import jax
import jax.numpy as jnp
from jax import lax
from jax.experimental import pallas as pl
from jax.experimental.pallas import tpu as pltpu
from jax.experimental.pallas import tpu_sc as plsc

N = 10000
E = 160000
F_IN = 256
HID = 512
G = 64
NP = 10240

NC, NS, L = 2, 16, 16
NW = NC * NS

_MESH = dict(core_axis_name="c", subcore_axis_name="s", num_cores=NC,
             num_subcores=NS)
_CP = pltpu.CompilerParams(needs_layout_passes=False)

F32 = jnp.float32
I32 = jnp.int32

_CH = 4096
_NCHUNK = 40
_EPAD = _CH * _NCHUNK
_CAP = _CH + 128


def _make_reduce(F, RS, nrange, gather, BK):
  npass = nrange // NW
  ACR = RS + 2
  NV = _CH // L

  scratch = [
      pltpu.VMEM((_CH,), I32),
      pltpu.VMEM((_CAP,), I32),
      pltpu.VMEM((ACR * F,), F32),
  ]
  if gather:
    scratch += [
        pltpu.VMEM((_CH,), I32),
        pltpu.VMEM((_CAP,), I32),
        pltpu.VMEM((BK, F), F32),
        pltpu.SemaphoreType.DMA,
    ]

  mesh = plsc.VectorSubcoreMesh(**_MESH)

  @functools.partial(
      pl.kernel,
      out_type=jax.ShapeDtypeStruct((nrange * RS * F,), F32),
      mesh=mesh,
      compiler_params=_CP,
      scratch_types=scratch,
  )
  def reduce_kernel(src_hbm, dst_hbm, t_hbm, out_hbm, dstv, relc, acc, *rest):
    s = lax.axis_index("s")
    c = lax.axis_index("c")
    w = c * NS + s
    if gather:
      srcv, srcc, rows, sem = rest
    iota = lax.iota(I32, L)
    ones = jnp.ones((L,), F32)

    def one_pass(p, carry0):
      base = (w + NW * p) * RS

      def zero(i, carry):
        acc[pl.ds(i * L, L)] = jnp.zeros((L,), F32)
        return carry

      lax.fori_loop(0, ACR * F // L, zero, jnp.int32(0))

      def chunk_body(ch, cnt_in):
        pltpu.sync_copy(dst_hbm.at[pl.ds(ch * _CH, _CH)], dstv)
        if gather:
          pltpu.sync_copy(src_hbm.at[pl.ds(ch * _CH, _CH)], srcv)

        def compact(i4, cnt):
          for r in range(4):
            i = i4 * 4 + r
            d = dstv[pl.ds(i * L, L)]
            rel = d - base
            m = (rel >= 0) & (rel < RS)
            nhit = plsc.all_reduce_population_count(m)[0]

            def dostore(cnt=cnt, m=m, rel=rel, i=i):
              mi = jnp.where(m, jnp.int32(1), jnp.int32(0))
              pos = cnt + plsc.cumsum(mi) - mi
              plsc.store_scatter(relc, [pos], rel, mask=m)
              if gather:
                sv = srcv[pl.ds(i * L, L)]
                plsc.store_scatter(srcc, [pos], sv, mask=m)

            pl.when(nhit > 0)(dostore)
            cnt = cnt + nhit
          return cnt

        cnt = lax.fori_loop(0, NV // 4, compact, cnt_in)

        if not gather:
          nb = (cnt + (BK - 1)) // BK
          for l in range(BK // L):
            lanes = cnt + l * L + iota
            m2 = lanes < nb * BK
            plsc.store_scatter(relc, [lanes], jnp.full((L,), RS, I32),
                               mask=m2)

          def proc(j, carry2):
            def onegrp(g, carry3):
              rv = relc[pl.ds(j * BK + g * L, L)]
              for l in range(L):
                plsc.addupdate(acc.at[pl.ds(rv[l] * 16, L)], ones)
              return carry3
            return lax.fori_loop(0, BK // L, onegrp, carry2)

          lax.fori_loop(0, nb, proc, jnp.int32(0))
          return jnp.int32(0)

        nfull = cnt // BK
        _proc_blocks(relc, srcc, acc, rows, sem, t_hbm, jnp.int32(0), nfull)
        left = cnt - nfull * BK
        for l in range(BK // L):
          lanes = l * L + iota
          m2 = lanes < left
          vals_r = relc[pl.ds(nfull * BK + l * L, L)]
          vals_s = srcc[pl.ds(nfull * BK + l * L, L)]
          plsc.store_scatter(relc, [lanes], vals_r, mask=m2)
          plsc.store_scatter(srcc, [lanes], vals_s, mask=m2)
        return left

      def _proc_blocks(relc, srcc, acc, rows, sem, t_hbm, lo, hi):
        def proc(j, carry2):
          pltpu.async_copy(t_hbm.at[srcc.at[pl.ds(j * BK, BK)]], rows,
                           sem).wait()

          def onegrp(g, carry3):
            rv = relc[pl.ds(j * BK + g * L, L)]
            for l in range(L):
              roff = rv[l] * F
              for k in range(0, F // L, 8):
                rr = [rows[g * L + l, pl.ds((k + u) * L, L)]
                      for u in range(8)]
                aa = [acc[pl.ds(roff + (k + u) * L, L)] for u in range(8)]
                for u in range(8):
                  acc[pl.ds(roff + (k + u) * L, L)] = aa[u] + rr[u]
            return carry3

          lax.fori_loop(0, BK // L, onegrp, jnp.int32(0))
          return carry2

        lax.fori_loop(lo, hi, proc, jnp.int32(0))

      cnt_last = lax.fori_loop(0, _NCHUNK, chunk_body, jnp.int32(0))
      if gather:
        nb2 = (cnt_last + (BK - 1)) // BK
        for l in range(BK // L):
          lanes = cnt_last + l * L + iota
          m2 = lanes < nb2 * BK
          plsc.store_scatter(relc, [lanes], jnp.full((L,), RS, I32),
                             mask=m2)
          plsc.store_scatter(srcc, [lanes], jnp.zeros((L,), I32), mask=m2)
        _proc_blocks(relc, srcc, acc, rows, sem, t_hbm, jnp.int32(0), nb2)
      pltpu.sync_copy(acc.at[pl.ds(0, RS * F)],
                      out_hbm.at[pl.ds(base * F, RS * F)])
      return carry0

    lax.fori_loop(0, npass, one_pass, jnp.int32(0))

  return reduce_kernel


_RS_DEG, _NR_DEG = 328, 32
_RS1, _NR1 = 328, 32
_RS2, _NR2 = 168, 64

_deg_red = _make_reduce(16, _RS_DEG, _NR_DEG, gather=False, BK=16)
_agg_fin = _make_reduce(F_IN, _RS1, _NR1, gather=True, BK=64)
_agg_hid = _make_reduce(HID, _RS2, _NR2, gather=True, BK=48)


_BM = 512
_GRID = NP // _BM


def _dinv_of(p_ref):
  deg = p_ref[:, 0:1] + 1.0
  return lax.rsqrt(deg)


def _tc_scale(p_ref, x_ref, o_ref):
  o_ref[...] = x_ref[...] * _dinv_of(p_ref)


def _tc_layer(a_ref, t_ref, p_ref, w_ref, b_ref, o_ref):
  dinv = _dinv_of(p_ref)
  u = (a_ref[...] + t_ref[...]) * dinv
  h = jnp.dot(u, w_ref[...], preferred_element_type=F32,
              precision=lax.Precision.HIGHEST)
  h = jnp.maximum(h + b_ref[...], 0.0)
  o_ref[...] = h * dinv


def _tc_pool(a_ref, t_ref, p_ref, w_ref, b_ref, batch_ref, o_ref,
             pooled, cnt2):
  i = pl.program_id(0)

  @pl.when(i == 0)
  def _():
    pooled[...] = jnp.zeros_like(pooled)
    cnt2[...] = jnp.zeros_like(cnt2)

  dinv = _dinv_of(p_ref)
  u = (a_ref[...] + t_ref[...]) * dinv
  h = jnp.dot(u, w_ref[...], preferred_element_type=F32,
              precision=lax.Precision.HIGHEST)
  h = jnp.maximum(h + b_ref[...], 0.0)

  gids = lax.broadcasted_iota(I32, (G, _BM), 0)
  oh = (batch_ref[0] == gids).astype(F32)
  pooled[...] += jnp.dot(oh, h, preferred_element_type=F32,
                         precision=lax.Precision.HIGHEST)
  cnt2[...] += oh

  @pl.when(i == _GRID - 1)
  def _():
    counts = jnp.sum(cnt2[...], axis=1, keepdims=True)
    o_ref[...] = pooled[...] / jnp.maximum(counts, 1.0)


def _row_spec(wd):
  return pl.BlockSpec((_BM, wd), lambda i: (i, 0))


def _full_spec(r, wd):
  return pl.BlockSpec((r, wd), lambda i: (0, 0))


def kernel(x, edge_index, batch, W1, b1, W2, b2):
  src = edge_index[0]
  dst = edge_index[1]
  srcp = jnp.pad(src, (0, _EPAD - E))
  dstp = jnp.pad(dst, (0, _EPAD - E), constant_values=NP)

  x_pad = jnp.pad(x, ((0, NP - N), (0, 0)))
  batch2 = jnp.pad(batch, (0, NP - N),
                   constant_values=G).reshape(_GRID, 1, _BM)
  b1r = b1.reshape(1, HID)
  b2r = b2.reshape(1, HID)
  dummy16 = jnp.zeros((1, 16), F32)

  p = _deg_red(srcp, dstp, dummy16).reshape(_NR_DEG * _RS_DEG, 16)

  t1 = pl.pallas_call(
      _tc_scale,
      grid=(_GRID,),
      in_specs=[_row_spec(16), _row_spec(F_IN)],
      out_specs=_row_spec(F_IN),
      out_shape=jax.ShapeDtypeStruct((NP, F_IN), F32),
  )(p, x_pad)

  a1 = _agg_fin(srcp, dstp, t1).reshape(_NR1 * _RS1, F_IN)

  t2 = pl.pallas_call(
      _tc_layer,
      grid=(_GRID,),
      in_specs=[_row_spec(F_IN), _row_spec(F_IN), _row_spec(16),
                _full_spec(F_IN, HID), _full_spec(1, HID)],
      out_specs=_row_spec(HID),
      out_shape=jax.ShapeDtypeStruct((NP, HID), F32),
  )(a1, t1, p, W1, b1r)

  a2 = _agg_hid(srcp, dstp, t2).reshape(_NR2 * _RS2, HID)

  out = pl.pallas_call(
      _tc_pool,
      grid=(_GRID,),
      in_specs=[_row_spec(HID), _row_spec(HID), _row_spec(16),
                _full_spec(HID, HID), _full_spec(1, HID),
                pl.BlockSpec((1, 1, _BM), lambda i: (i, 0, 0))],
      out_specs=_full_spec(G, HID),
      out_shape=jax.ShapeDtypeStruct((G, HID), F32),
      scratch_shapes=[pltpu.VMEM((G, HID), F32), pltpu.VMEM((G, _BM), F32)],
  )(a2, t2, p, W2, b2r, batch2)

  return out
